# Initial kernel scaffold; baseline (speedup 1.0000x reference)
import jax
import jax.numpy as jnp
from jax.experimental import pallas as pl


def kernel(x):
    raise NotImplementedError("write your pallas kernel here")


# speedup vs baseline: 12.0246x; 12.0246x over previous
import functools

import jax
import jax.numpy as jnp
from jax import lax
from jax.experimental import pallas as pl
from jax.experimental.pallas import tpu as pltpu
from jax.experimental.pallas import tpu_sc as plsc

N = 128 * 32768
K = N // 2
RANK = K - 1
NC, NS, L = 2, 16, 16
NW = NC * NS
PT = N // NW
CH = 8192
NB1, SH1 = 2048, 20
NB2, SH2 = 1024, 10
NB3, SH3 = 1024, 0
MASK31 = 0x7FFFFFFF

_mesh = plsc.VectorSubcoreMesh(core_axis_name="c", subcore_axis_name="s")


def _lanes():
    return lax.iota(jnp.int32, L)


def _wid():
    return lax.axis_index("s") * NC + lax.axis_index("c")


def _zero(ref, nwords):
    z = jnp.zeros((L,), jnp.int32)

    def body(i, _):
        ref[pl.ds(i * L, L)] = z
        return 0

    lax.fori_loop(0, nwords // L, body, 0)


def _accum_hist(x_hbm, base, buf, lh, nb, shift, pred_shift, pred_val):
    laneoff = _lanes() * nb
    ones = jnp.ones((L,), jnp.int32)

    def chunk(ci, _):
        pltpu.sync_copy(x_hbm.at[pl.ds(base + ci * CH, CH)], buf)

        def vec(i, _):
            v = buf[pl.ds(i * L, L)]
            u = lax.bitcast_convert_type(v, jnp.int32) & MASK31
            b = (u >> shift) & (nb - 1)
            idx = laneoff + b
            if pred_shift is None:
                plsc.addupdate_scatter(lh, [idx], ones)
            else:
                pred = (u >> pred_shift) == jnp.broadcast_to(pred_val, (L,))
                plsc.addupdate_scatter(lh, [idx], ones, mask=pred)
            return 0

        lax.fori_loop(0, CH // L, vec, 0)
        return 0

    lax.fori_loop(0, PT // CH, chunk, 0)


def _lane_reduce(lh, local, nb):
    def group(g, _):
        def one_lane(l, acc):
            return acc + lh[pl.ds(l * nb + g * L, L)]

        acc = lax.fori_loop(0, L, one_lane, jnp.zeros((L,), jnp.int32))
        local[pl.ds(g * L, L)] = acc
        return 0

    lax.fori_loop(0, nb // L, group, 0)


def _select(hbuf, nb, rank):
    lane = _lanes()

    def group(g, carry):
        run, pacc, excacc = carry

        def one_tile(t, acc):
            return acc + hbuf[t, pl.ds(g * L, L)]

        v = lax.fori_loop(0, NW, one_tile, jnp.zeros((L,), jnp.int32))
        inc = plsc.cumsum(v) + run
        exc = inc - v
        pred = (exc <= rank) & (inc > rank)
        pacc = pacc + jnp.sum(jnp.where(pred, g * L + lane, 0))
        excacc = excacc + jnp.sum(jnp.where(pred, exc, 0))
        run = run + jnp.sum(v)
        return run, pacc, excacc

    init = (jnp.int32(0), jnp.int32(0), jnp.int32(0))
    _, p, exc = lax.fori_loop(0, nb // L, group, init)
    return p, rank - exc


def _read_pair(selbuf):
    lane = _lanes()
    v = selbuf[...]
    a = jnp.sum(jnp.where(lane == 0, v, 0))
    b = jnp.sum(jnp.where(lane == 1, v, 0))
    return a, b


def _write_pair(selbuf, out_hbm, a, b):
    lane = _lanes()
    selbuf[...] = jnp.where(lane == 0, a, 0) + jnp.where(lane == 1, b, 0)
    pltpu.sync_copy(selbuf, out_hbm)


@functools.partial(
    pl.kernel,
    out_type=jax.ShapeDtypeStruct((NW, NB1), jnp.int32),
    mesh=_mesh,
    compiler_params=pltpu.CompilerParams(needs_layout_passes=False),
    scratch_types=[
        pltpu.VMEM((L * NB1,), jnp.int32),
        pltpu.VMEM((NB1,), jnp.int32),
        pltpu.VMEM((CH,), jnp.float32),
    ],
)
def _k1(x_hbm, h1_hbm, lh, local, buf):
    wid = _wid()
    _zero(lh, L * NB1)
    _accum_hist(x_hbm, wid * PT, buf, lh, NB1, SH1, None, None)
    _lane_reduce(lh, local, NB1)
    pltpu.sync_copy(local, h1_hbm.at[wid])


@functools.partial(
    pl.kernel,
    out_type=(
        jax.ShapeDtypeStruct((NW, NB2), jnp.int32),
        jax.ShapeDtypeStruct((L,), jnp.int32),
    ),
    mesh=_mesh,
    compiler_params=pltpu.CompilerParams(needs_layout_passes=False),
    scratch_types=[
        pltpu.VMEM((NW, NB1), jnp.int32),
        pltpu.VMEM((L * NB2,), jnp.int32),
        pltpu.VMEM((NB2,), jnp.int32),
        pltpu.VMEM((CH,), jnp.float32),
        pltpu.VMEM((L,), jnp.int32),
    ],
)
def _k2(x_hbm, h1_hbm, h2_hbm, sel_hbm, h1buf, lh, local, buf, selbuf):
    wid = _wid()
    pltpu.sync_copy(h1_hbm, h1buf)
    p1, r1 = _select(h1buf, NB1, jnp.int32(RANK))
    _zero(lh, L * NB2)
    _accum_hist(x_hbm, wid * PT, buf, lh, NB2, SH2, SH1, p1)
    _lane_reduce(lh, local, NB2)
    pltpu.sync_copy(local, h2_hbm.at[wid])

    @pl.when(wid == 0)
    def _():
        _write_pair(selbuf, sel_hbm, p1, r1)


@functools.partial(
    pl.kernel,
    out_type=(
        jax.ShapeDtypeStruct((NW, NB3), jnp.int32),
        jax.ShapeDtypeStruct((L,), jnp.int32),
    ),
    mesh=_mesh,
    compiler_params=pltpu.CompilerParams(needs_layout_passes=False),
    scratch_types=[
        pltpu.VMEM((NW, NB2), jnp.int32),
        pltpu.VMEM((L * NB3,), jnp.int32),
        pltpu.VMEM((NB3,), jnp.int32),
        pltpu.VMEM((CH,), jnp.float32),
        pltpu.VMEM((L,), jnp.int32),
    ],
)
def _k3(x_hbm, h2_hbm, sel1_hbm, h3_hbm, sel2_hbm, h2buf, lh, local, buf, selbuf):
    wid = _wid()
    pltpu.sync_copy(sel1_hbm, selbuf)
    p1, r1 = _read_pair(selbuf)
    pltpu.sync_copy(h2_hbm, h2buf)
    p2, r2 = _select(h2buf, NB2, r1)
    prefix21 = (p1 << 10) | p2
    _zero(lh, L * NB3)
    _accum_hist(x_hbm, wid * PT, buf, lh, NB3, SH3, SH2, prefix21)
    _lane_reduce(lh, local, NB3)
    pltpu.sync_copy(local, h3_hbm.at[wid])

    @pl.when(wid == 0)
    def _():
        _write_pair(selbuf, sel2_hbm, prefix21, r2)


@functools.partial(
    pl.kernel,
    out_type=jax.ShapeDtypeStruct((L,), jnp.int32),
    mesh=_mesh,
    compiler_params=pltpu.CompilerParams(needs_layout_passes=False),
    scratch_types=[
        pltpu.VMEM((NW, NB3), jnp.int32),
        pltpu.VMEM((L,), jnp.int32),
    ],
)
def _k4(h3_hbm, sel2_hbm, thr_hbm, h3buf, selbuf):
    wid = _wid()

    @pl.when(wid == 0)
    def _():
        pltpu.sync_copy(sel2_hbm, selbuf)
        prefix21, r2 = _read_pair(selbuf)
        pltpu.sync_copy(h3_hbm, h3buf)
        p3, _r3 = _select(h3buf, NB3, r2)
        thr = (prefix21 << 10) | p3
        selbuf[...] = jnp.broadcast_to(thr, (L,))
        pltpu.sync_copy(selbuf, thr_hbm)


def _mask_body(thr_ref, x_ref, o_ref):
    t = thr_ref[0, 0]
    bits = lax.bitcast_convert_type(x_ref[...], jnp.int32)
    u = bits & MASK31
    o_ref[...] = jnp.where(u > t, x_ref[...], 0.0)


def _apply_mask(x, thr):
    rows = x.shape[0]
    rb = 16
    return pl.pallas_call(
        _mask_body,
        grid=(rows // rb,),
        in_specs=[
            pl.BlockSpec(memory_space=pltpu.SMEM),
            pl.BlockSpec((rb, x.shape[1]), lambda i: (i, 0)),
        ],
        out_specs=pl.BlockSpec((rb, x.shape[1]), lambda i: (i, 0)),
        out_shape=jax.ShapeDtypeStruct(x.shape, x.dtype),
    )(thr, x)


@jax.jit
def kernel(x):
    xf = x.reshape(-1)
    h1 = _k1(xf)
    h2, s1 = _k2(xf, h1)
    h3, s2 = _k3(xf, h2, s1)
    thr = _k4(h3, s2)
    return _apply_mask(x, thr.reshape(1, L))


# speedup vs baseline: 16.6408x; 1.3839x over previous
import functools

import jax
import jax.numpy as jnp
from jax import lax
from jax.experimental import pallas as pl
from jax.experimental.pallas import tpu as pltpu
from jax.experimental.pallas import tpu_sc as plsc

N = 128 * 32768
K = N // 2
RANK = K - 1
NC, NS, L = 2, 16, 16
NW = NC * NS
PT = N // NW
CH = 16384
UN = 8
NB1, SH1 = 2048, 20
NB2, SH2 = 1024, 10
NB3, SH3 = 1024, 0
MASK31 = 0x7FFFFFFF

_mesh = plsc.VectorSubcoreMesh(core_axis_name="c", subcore_axis_name="s")
_params = pltpu.CompilerParams(needs_layout_passes=False)


def _lanes():
    return lax.iota(jnp.int32, L)


def _wid():
    return lax.axis_index("s") * NC + lax.axis_index("c")


def _zero(ref, nwords):
    z = jnp.zeros((L,), jnp.int32)

    def body(i, _):
        base = i * (L * UN)
        for j in range(UN):
            ref[pl.ds(base + j * L, L)] = z
        return 0

    lax.fori_loop(0, nwords // (L * UN), body, 0)


def _accum_hist(x_hbm, base, bufs, sems, lh, nb, shift, pred_shift, pred_val):
    laneoff = _lanes() * nb
    ones = jnp.ones((L,), jnp.int32)
    nch = PT // CH

    def compute(buf):
        def vec(i, _):
            vbase = i * (L * UN)
            for j in range(UN):
                v = buf[pl.ds(vbase + j * L, L)]
                u = lax.bitcast_convert_type(v, jnp.int32) & MASK31
                b = (u >> shift) & (nb - 1)
                idx = laneoff + b
                if pred_shift is None:
                    plsc.addupdate_scatter(lh, [idx], ones)
                else:
                    pred = (u >> pred_shift) == jnp.broadcast_to(pred_val, (L,))
                    plsc.addupdate_scatter(lh, [idx], ones, mask=pred)
            return 0

        lax.fori_loop(0, CH // (L * UN), vec, 0)

    def start(ci):
        src = x_hbm.at[pl.ds(base + ci * CH, CH)]
        return pltpu.async_copy(src, bufs[ci % 2], sems[ci % 2])

    pending = start(0)
    for ci in range(nch):
        nxt = start(ci + 1) if ci + 1 < nch else None
        pending.wait()
        compute(bufs[ci % 2])
        pending = nxt


def _lane_reduce(lh, local, nb):
    def group(g, _):
        off = g * L
        acc = lh[pl.ds(off, L)]
        for l in range(1, L):
            acc = acc + lh[pl.ds(l * nb + off, L)]
        local[pl.ds(off, L)] = acc
        return 0

    lax.fori_loop(0, nb // L, group, 0)


def _select(hbuf, nb, rank):
    lane = _lanes()

    def group(g, carry):
        run, pacc, excacc = carry
        off = g * L
        v = hbuf[0, pl.ds(off, L)]
        for t in range(1, NW):
            v = v + hbuf[t, pl.ds(off, L)]
        inc = plsc.cumsum(v) + run
        exc = inc - v
        pred = (exc <= rank) & (inc > rank)
        pacc = pacc + jnp.sum(jnp.where(pred, off + lane, 0))
        excacc = excacc + jnp.sum(jnp.where(pred, exc, 0))
        run = run + jnp.sum(v)
        return run, pacc, excacc

    init = (jnp.int32(0), jnp.int32(0), jnp.int32(0))
    _, p, exc = lax.fori_loop(0, nb // L, group, init)
    return p, rank - exc


def _read_pair(selbuf):
    lane = _lanes()
    v = selbuf[...]
    a = jnp.sum(jnp.where(lane == 0, v, 0))
    b = jnp.sum(jnp.where(lane == 1, v, 0))
    return a, b


def _write_pair(selbuf, out_hbm, a, b):
    lane = _lanes()
    selbuf[...] = jnp.where(lane == 0, a, 0) + jnp.where(lane == 1, b, 0)
    pltpu.sync_copy(selbuf, out_hbm)


@functools.partial(
    pl.kernel,
    out_type=jax.ShapeDtypeStruct((NW, NB1), jnp.int32),
    mesh=_mesh,
    compiler_params=_params,
    scratch_types=[
        pltpu.VMEM((L * NB1,), jnp.int32),
        pltpu.VMEM((NB1,), jnp.int32),
        pltpu.VMEM((CH,), jnp.float32),
        pltpu.VMEM((CH,), jnp.float32),
        pltpu.SemaphoreType.DMA,
        pltpu.SemaphoreType.DMA,
    ],
)
def _k1(x_hbm, h1_hbm, lh, local, b0, b1, s0, s1):
    wid = _wid()
    _zero(lh, L * NB1)
    _accum_hist(x_hbm, wid * PT, (b0, b1), (s0, s1), lh, NB1, SH1, None, None)
    _lane_reduce(lh, local, NB1)
    pltpu.sync_copy(local, h1_hbm.at[wid])


@functools.partial(
    pl.kernel,
    out_type=(
        jax.ShapeDtypeStruct((NW, NB2), jnp.int32),
        jax.ShapeDtypeStruct((L,), jnp.int32),
    ),
    mesh=_mesh,
    compiler_params=_params,
    scratch_types=[
        pltpu.VMEM((NW, NB1), jnp.int32),
        pltpu.VMEM((L * NB2,), jnp.int32),
        pltpu.VMEM((NB2,), jnp.int32),
        pltpu.VMEM((CH,), jnp.float32),
        pltpu.VMEM((CH,), jnp.float32),
        pltpu.VMEM((L,), jnp.int32),
        pltpu.SemaphoreType.DMA,
        pltpu.SemaphoreType.DMA,
    ],
)
def _k2(x_hbm, h1_hbm, h2_hbm, sel_hbm, h1buf, lh, local, b0, b1, selbuf, s0, s1):
    wid = _wid()
    pltpu.sync_copy(h1_hbm, h1buf)
    p1, r1 = _select(h1buf, NB1, jnp.int32(RANK))
    _zero(lh, L * NB2)
    _accum_hist(x_hbm, wid * PT, (b0, b1), (s0, s1), lh, NB2, SH2, SH1, p1)
    _lane_reduce(lh, local, NB2)
    pltpu.sync_copy(local, h2_hbm.at[wid])

    @pl.when(wid == 0)
    def _():
        _write_pair(selbuf, sel_hbm, p1, r1)


@functools.partial(
    pl.kernel,
    out_type=(
        jax.ShapeDtypeStruct((NW, NB3), jnp.int32),
        jax.ShapeDtypeStruct((L,), jnp.int32),
    ),
    mesh=_mesh,
    compiler_params=_params,
    scratch_types=[
        pltpu.VMEM((NW, NB2), jnp.int32),
        pltpu.VMEM((L * NB3,), jnp.int32),
        pltpu.VMEM((NB3,), jnp.int32),
        pltpu.VMEM((CH,), jnp.float32),
        pltpu.VMEM((CH,), jnp.float32),
        pltpu.VMEM((L,), jnp.int32),
        pltpu.SemaphoreType.DMA,
        pltpu.SemaphoreType.DMA,
    ],
)
def _k3(x_hbm, h2_hbm, sel1_hbm, h3_hbm, sel2_hbm, h2buf, lh, local, b0, b1,
        selbuf, s0, s1):
    wid = _wid()
    pltpu.sync_copy(sel1_hbm, selbuf)
    p1, r1 = _read_pair(selbuf)
    pltpu.sync_copy(h2_hbm, h2buf)
    p2, r2 = _select(h2buf, NB2, r1)
    prefix21 = (p1 << 10) | p2
    _zero(lh, L * NB3)
    _accum_hist(x_hbm, wid * PT, (b0, b1), (s0, s1), lh, NB3, SH3, SH2, prefix21)
    _lane_reduce(lh, local, NB3)
    pltpu.sync_copy(local, h3_hbm.at[wid])

    @pl.when(wid == 0)
    def _():
        _write_pair(selbuf, sel2_hbm, prefix21, r2)


def _mask_body(sel_ref, x_ref, h3_ref, o_ref, tscr):
    @pl.when(pl.program_id(0) == 0)
    def _():
        m = jnp.sum(h3_ref[...].astype(jnp.float32), axis=0, keepdims=True)
        ii = lax.broadcasted_iota(jnp.int32, (NB3, NB3), 0)
        jj = lax.broadcasted_iota(jnp.int32, (NB3, NB3), 1)
        tri = jnp.where(ii < jj, 1.0, 0.0)
        exc = jnp.dot(m, tri, preferred_element_type=jnp.float32)
        inc = exc + m
        r2 = sel_ref[0, 1].astype(jnp.float32)
        pred = (exc <= r2) & (inc > r2)
        bidx = lax.broadcasted_iota(jnp.int32, (1, NB3), 1)
        p3 = jnp.sum(jnp.where(pred, bidx, 0))
        tscr[0, 0] = (sel_ref[0, 0] << 10) | p3

    t = tscr[0, 0]
    bits = lax.bitcast_convert_type(x_ref[...], jnp.int32)
    u = bits & MASK31
    o_ref[...] = jnp.where(u > t, x_ref[...], 0.0)


def _apply_mask(x, h3, sel2):
    rows = x.shape[0]
    rb = 16
    return pl.pallas_call(
        _mask_body,
        grid=(rows // rb,),
        in_specs=[
            pl.BlockSpec(memory_space=pltpu.SMEM),
            pl.BlockSpec((rb, x.shape[1]), lambda i: (i, 0)),
            pl.BlockSpec((NW, NB3), lambda i: (0, 0)),
        ],
        out_specs=pl.BlockSpec((rb, x.shape[1]), lambda i: (i, 0)),
        out_shape=jax.ShapeDtypeStruct(x.shape, x.dtype),
        scratch_shapes=[pltpu.SMEM((1, 1), jnp.int32)],
    )(sel2.reshape(1, L), x, h3)


@jax.jit
def kernel(x):
    xf = x.reshape(-1)
    h1 = _k1(xf)
    h2, s1 = _k2(xf, h1)
    h3, s2 = _k3(xf, h2, s1)
    return _apply_mask(x, h3, s2)


# speedup vs baseline: 38.3651x; 2.3055x over previous
import functools

import jax
import jax.numpy as jnp
from jax import lax
from jax.experimental import pallas as pl
from jax.experimental.pallas import tpu as pltpu
from jax.experimental.pallas import tpu_sc as plsc

ROWS, COLS = 128, 32768
N = ROWS * COLS
K = N // 2
RANK = K - 1
NC, NS, L = 2, 16, 16
NW = NC * NS
RB = 8
CB = COLS // 2
CW = 2048
NCH = CB // CW
UN = 16
NB1, SH1 = 2048, 20
NB2, SH2 = 1024, 10
NB3, SH3 = 1024, 0
MASK31 = 0x7FFFFFFF

_mesh = plsc.VectorSubcoreMesh(core_axis_name="c", subcore_axis_name="s")
_params = pltpu.CompilerParams(
    needs_layout_passes=False, use_tc_tiling_on_sc=True)


def _lanes():
    return lax.iota(jnp.int32, L)


def _wid():
    return lax.axis_index("s") * NC + lax.axis_index("c")


def _zero(ref, nwords):
    z = jnp.zeros((L,), jnp.int32)

    def body(i, _):
        base = i * (L * UN)
        for j in range(UN):
            ref[pl.ds(base + j * L, L)] = z
        return 0

    lax.fori_loop(0, nwords // (L * UN), body, 0)


def _start_chunk(x_hbm, wid, bufs, sems, ci):
    r0 = (wid // 2) * RB
    c0 = (wid % 2) * CB + ci * CW
    src = x_hbm.at[pl.ds(r0, RB), pl.ds(c0, CW)]
    return pltpu.async_copy(src, bufs[ci % 2], sems[ci % 2])


def _accum_hist(x_hbm, wid, bufs, sems, pend, lh, nb, shift, pred_shift,
                pred_val):
    laneoff = _lanes() * nb
    ones = jnp.ones((L,), jnp.int32)

    def compute(buf):
        def vec(i, _):
            vbase = i * (L * UN)
            for r in range(RB):
                idxs = []
                preds = []
                for j in range(UN):
                    v = buf[r, pl.ds(vbase + j * L, L)]
                    u = lax.bitcast_convert_type(v, jnp.int32) & MASK31
                    b = (u >> shift) & (nb - 1)
                    idxs.append(laneoff + b)
                    if pred_shift is not None:
                        preds.append(
                            (u >> pred_shift) == jnp.broadcast_to(pred_val, (L,)))
                for j in range(UN):
                    if pred_shift is None:
                        plsc.addupdate_scatter(lh, [idxs[j]], ones)
                    else:
                        plsc.addupdate_scatter(lh, [idxs[j]], ones,
                                               mask=preds[j])
            return 0

        lax.fori_loop(0, CW // (L * UN), vec, 0)

    for ci in range(NCH):
        pend[ci % 2].wait()
        compute(bufs[ci % 2])
        if ci + 2 < NCH:
            pend[ci % 2] = _start_chunk(x_hbm, wid, bufs, sems, ci + 2)


def _lane_reduce(lh, local, nb):
    def group(g, _):
        off = g * L
        acc = lh[pl.ds(off, L)]
        for l in range(1, L):
            acc = acc + lh[pl.ds(l * nb + off, L)]
        local[pl.ds(off, L)] = acc
        return 0

    lax.fori_loop(0, nb // L, group, 0)


def _select(hbuf, nb, rank):
    lane = _lanes()

    def group(g, carry):
        run, pacc, excacc = carry
        off = g * L
        v = hbuf[pl.ds(off, L)]
        for t in range(1, NW):
            v = v + hbuf[pl.ds(t * nb + off, L)]
        inc = plsc.cumsum(v) + run
        exc = inc - v
        pred = (exc <= rank) & (inc > rank)
        pacc = pacc + jnp.sum(jnp.where(pred, off + lane, 0))
        excacc = excacc + jnp.sum(jnp.where(pred, exc, 0))
        run = run + jnp.sum(v)
        return run, pacc, excacc

    init = (jnp.int32(0), jnp.int32(0), jnp.int32(0))
    _, p, exc = lax.fori_loop(0, nb // L, group, init)
    return p, rank - exc


def _read_pair(selbuf):
    lane = _lanes()
    v = selbuf[...]
    a = jnp.sum(jnp.where(lane == 0, v, 0))
    b = jnp.sum(jnp.where(lane == 1, v, 0))
    return a, b


def _write_pair(selbuf, out_hbm, a, b):
    lane = _lanes()
    selbuf[...] = jnp.where(lane == 0, a, 0) + jnp.where(lane == 1, b, 0)
    pltpu.sync_copy(selbuf, out_hbm)


@functools.partial(
    pl.kernel,
    out_type=jax.ShapeDtypeStruct((NW * NB1,), jnp.int32),
    mesh=_mesh,
    compiler_params=_params,
    scratch_types=[
        pltpu.VMEM((L * NB1,), jnp.int32),
        pltpu.VMEM((NB1,), jnp.int32),
        pltpu.VMEM((RB, CW), jnp.float32),
        pltpu.VMEM((RB, CW), jnp.float32),
        pltpu.SemaphoreType.DMA,
        pltpu.SemaphoreType.DMA,
    ],
)
def _k1(x_hbm, h1_hbm, lh, local, b0, b1, s0, s1):
    wid = _wid()
    bufs, sems = (b0, b1), (s0, s1)
    pend = [_start_chunk(x_hbm, wid, bufs, sems, 0),
            _start_chunk(x_hbm, wid, bufs, sems, 1)]
    _zero(lh, L * NB1)
    _accum_hist(x_hbm, wid, bufs, sems, pend, lh, NB1, SH1, None, None)
    _lane_reduce(lh, local, NB1)
    pltpu.sync_copy(local, h1_hbm.at[pl.ds(wid * NB1, NB1)])


@functools.partial(
    pl.kernel,
    out_type=(
        jax.ShapeDtypeStruct((NW * NB2,), jnp.int32),
        jax.ShapeDtypeStruct((L,), jnp.int32),
    ),
    mesh=_mesh,
    compiler_params=_params,
    scratch_types=[
        pltpu.VMEM((NW * NB1,), jnp.int32),
        pltpu.VMEM((L * NB2,), jnp.int32),
        pltpu.VMEM((NB2,), jnp.int32),
        pltpu.VMEM((RB, CW), jnp.float32),
        pltpu.VMEM((RB, CW), jnp.float32),
        pltpu.VMEM((L,), jnp.int32),
        pltpu.SemaphoreType.DMA,
        pltpu.SemaphoreType.DMA,
    ],
)
def _k2(x_hbm, h1_hbm, h2_hbm, sel_hbm, h1buf, lh, local, b0, b1, selbuf,
        s0, s1):
    wid = _wid()
    bufs, sems = (b0, b1), (s0, s1)
    pend = [_start_chunk(x_hbm, wid, bufs, sems, 0),
            _start_chunk(x_hbm, wid, bufs, sems, 1)]
    pltpu.sync_copy(h1_hbm, h1buf)
    p1, r1 = _select(h1buf, NB1, jnp.int32(RANK))
    _zero(lh, L * NB2)
    _accum_hist(x_hbm, wid, bufs, sems, pend, lh, NB2, SH2, SH1, p1)
    _lane_reduce(lh, local, NB2)
    pltpu.sync_copy(local, h2_hbm.at[pl.ds(wid * NB2, NB2)])

    @pl.when(wid == 0)
    def _():
        _write_pair(selbuf, sel_hbm, p1, r1)


@functools.partial(
    pl.kernel,
    out_type=(
        jax.ShapeDtypeStruct((NW * NB3,), jnp.int32),
        jax.ShapeDtypeStruct((L,), jnp.int32),
    ),
    mesh=_mesh,
    compiler_params=_params,
    scratch_types=[
        pltpu.VMEM((NW * NB2,), jnp.int32),
        pltpu.VMEM((L * NB3,), jnp.int32),
        pltpu.VMEM((NB3,), jnp.int32),
        pltpu.VMEM((RB, CW), jnp.float32),
        pltpu.VMEM((RB, CW), jnp.float32),
        pltpu.VMEM((L,), jnp.int32),
        pltpu.SemaphoreType.DMA,
        pltpu.SemaphoreType.DMA,
    ],
)
def _k3(x_hbm, h2_hbm, sel1_hbm, h3_hbm, sel2_hbm, h2buf, lh, local, b0, b1,
        selbuf, s0, s1):
    wid = _wid()
    bufs, sems = (b0, b1), (s0, s1)
    pend = [_start_chunk(x_hbm, wid, bufs, sems, 0),
            _start_chunk(x_hbm, wid, bufs, sems, 1)]
    pltpu.sync_copy(sel1_hbm, selbuf)
    p1, r1 = _read_pair(selbuf)
    pltpu.sync_copy(h2_hbm, h2buf)
    p2, r2 = _select(h2buf, NB2, r1)
    prefix21 = (p1 << 10) | p2
    _zero(lh, L * NB3)
    _accum_hist(x_hbm, wid, bufs, sems, pend, lh, NB3, SH3, SH2, prefix21)
    _lane_reduce(lh, local, NB3)
    pltpu.sync_copy(local, h3_hbm.at[pl.ds(wid * NB3, NB3)])

    @pl.when(wid == 0)
    def _():
        _write_pair(selbuf, sel2_hbm, prefix21, r2)


def _mask_body(sel_ref, x_ref, h3_ref, tri_ref, o_ref, tscr):
    @pl.when(pl.program_id(0) == 0)
    def _():
        m = jnp.sum(h3_ref[...].astype(jnp.float32), axis=0, keepdims=True)
        exc = jnp.dot(m, tri_ref[...], preferred_element_type=jnp.float32)
        inc = exc + m
        r2 = sel_ref[0, 1].astype(jnp.float32)
        pred = (exc <= r2) & (inc > r2)
        bidx = lax.broadcasted_iota(jnp.int32, (1, NB3), 1)
        p3 = jnp.sum(jnp.where(pred, bidx, 0))
        tscr[0, 0] = (sel_ref[0, 0] << 10) | p3

    t = tscr[0, 0]
    bits = lax.bitcast_convert_type(x_ref[...], jnp.int32)
    u = bits & MASK31
    o_ref[...] = jnp.where(u > t, x_ref[...], 0.0)


def _apply_mask(x, h3, sel2):
    rows = x.shape[0]
    rb = 16
    tri = (jnp.arange(NB3, dtype=jnp.int32)[:, None]
           < jnp.arange(NB3, dtype=jnp.int32)[None, :]).astype(jnp.float32)
    return pl.pallas_call(
        _mask_body,
        grid=(rows // rb,),
        in_specs=[
            pl.BlockSpec(memory_space=pltpu.SMEM),
            pl.BlockSpec((rb, x.shape[1]), lambda i: (i, 0)),
            pl.BlockSpec((NW, NB3), lambda i: (0, 0)),
            pl.BlockSpec((NB3, NB3), lambda i: (0, 0)),
        ],
        out_specs=pl.BlockSpec((rb, x.shape[1]), lambda i: (i, 0)),
        out_shape=jax.ShapeDtypeStruct(x.shape, x.dtype),
        scratch_shapes=[pltpu.SMEM((1, 1), jnp.int32)],
    )(sel2.reshape(1, L), x, h3, tri)


@jax.jit
def kernel(x):
    h1 = _k1(x)
    h2, s1 = _k2(x, h1)
    h3, s2 = _k3(x, h2, s1)
    return _apply_mask(x, h3.reshape(NW, NB3), s2)


# speedup vs baseline: 39.4351x; 1.0279x over previous
import functools

import jax
import jax.numpy as jnp
from jax import lax
from jax.experimental import pallas as pl
from jax.experimental.pallas import tpu as pltpu
from jax.experimental.pallas import tpu_sc as plsc

ROWS, COLS = 128, 32768
N = ROWS * COLS
K = N // 2
RANK = K - 1
NC, NS, L = 2, 16, 16
NW = NC * NS
RB = 8
CB = COLS // 2
CW = 2048
NCH = CB // CW
UN = 16
NB1, SH1 = 2048, 20
NB2, SH2 = 1024, 10
NB3, SH3 = 1024, 0
MASK31 = 0x7FFFFFFF

_mesh = plsc.VectorSubcoreMesh(core_axis_name="c", subcore_axis_name="s")
_params = pltpu.CompilerParams(
    needs_layout_passes=False, use_tc_tiling_on_sc=True)


def _lanes():
    return lax.iota(jnp.int32, L)


def _wid():
    return lax.axis_index("s") * NC + lax.axis_index("c")


def _zero(ref, nwords):
    z = jnp.zeros((L,), jnp.int32)

    def body(i, _):
        base = i * (L * UN)
        for j in range(UN):
            ref[pl.ds(base + j * L, L)] = z
        return 0

    lax.fori_loop(0, nwords // (L * UN), body, 0)


def _start_chunk(x_hbm, wid, bufs, sems, ci):
    r0 = (wid // 2) * RB
    c0 = (wid % 2) * CB + ci * CW
    src = x_hbm.at[pl.ds(r0, RB), pl.ds(c0, CW)]
    return pltpu.async_copy(src, bufs[ci % 2], sems[ci % 2])


def _accum_hist(x_hbm, wid, bufs, sems, pend, lh, nb, shift, pred_shift,
                pred_val):
    laneoff = _lanes() * nb
    ones = jnp.ones((L,), jnp.int32)

    def compute(buf):
        def vec(i, _):
            vbase = i * (L * UN)
            for r in range(RB):
                idxs = []
                preds = []
                for j in range(UN):
                    v = buf[r, pl.ds(vbase + j * L, L)]
                    u = lax.bitcast_convert_type(v, jnp.int32) & MASK31
                    b = (u >> shift) & (nb - 1)
                    idxs.append(laneoff + b)
                    if pred_shift is not None:
                        preds.append(
                            (u >> pred_shift) == jnp.broadcast_to(pred_val, (L,)))
                for j in range(UN):
                    if pred_shift is None:
                        plsc.addupdate_scatter(lh, [idxs[j]], ones)
                    else:
                        plsc.addupdate_scatter(lh, [idxs[j]], ones,
                                               mask=preds[j])
            return 0

        lax.fori_loop(0, CW // (L * UN), vec, 0)

    for ci in range(NCH):
        pend[ci % 2].wait()
        compute(bufs[ci % 2])
        if ci + 2 < NCH:
            pend[ci % 2] = _start_chunk(x_hbm, wid, bufs, sems, ci + 2)


def _lane_reduce(lh, local, nb):
    def group(g, _):
        off = g * L
        acc = lh[pl.ds(off, L)]
        for l in range(1, L):
            acc = acc + lh[pl.ds(l * nb + off, L)]
        local[pl.ds(off, L)] = acc
        return 0

    lax.fori_loop(0, nb // L, group, 0)


def _sc_merge_epilogue(local, sh, tmp, out_hbm, nb):
    sid = lax.axis_index("s")
    pltpu.sync_copy(local, sh.at[sid])
    plsc.subcore_barrier()

    @pl.when(sid == 0)
    def _():
        pltpu.sync_copy(sh, tmp)

        def group(g, _):
            off = g * L
            acc = tmp[0, pl.ds(off, L)]
            for t in range(1, NS):
                acc = acc + tmp[t, pl.ds(off, L)]
            local[pl.ds(off, L)] = acc
            return 0

        lax.fori_loop(0, nb // L, group, 0)
        scid = lax.axis_index("c")
        pltpu.sync_copy(local, out_hbm.at[pl.ds(scid * nb, nb)])


def _select(hbuf, nb, rank):
    lane = _lanes()

    def group(g, carry):
        run, pacc, excacc = carry
        off = g * L
        v = hbuf[pl.ds(off, L)] + hbuf[pl.ds(nb + off, L)]
        inc = plsc.cumsum(v) + run
        exc = inc - v
        pred = (exc <= rank) & (inc > rank)
        pacc = pacc + jnp.sum(jnp.where(pred, off + lane, 0))
        excacc = excacc + jnp.sum(jnp.where(pred, exc, 0))
        run = run + jnp.sum(v)
        return run, pacc, excacc

    init = (jnp.int32(0), jnp.int32(0), jnp.int32(0))
    _, p, exc = lax.fori_loop(0, nb // L, group, init)
    return p, rank - exc


def _read_pair(selbuf):
    lane = _lanes()
    v = selbuf[...]
    a = jnp.sum(jnp.where(lane == 0, v, 0))
    b = jnp.sum(jnp.where(lane == 1, v, 0))
    return a, b


def _write_pair(selbuf, out_hbm, a, b):
    lane = _lanes()
    selbuf[...] = jnp.where(lane == 0, a, 0) + jnp.where(lane == 1, b, 0)
    pltpu.sync_copy(selbuf, out_hbm)


@functools.partial(
    pl.kernel,
    out_type=jax.ShapeDtypeStruct((2 * NB1,), jnp.int32),
    mesh=_mesh,
    compiler_params=_params,
    scratch_types=[
        pltpu.VMEM((L * NB1,), jnp.int32),
        pltpu.VMEM((NB1,), jnp.int32),
        pltpu.VMEM((RB, CW), jnp.float32),
        pltpu.VMEM((RB, CW), jnp.float32),
        pltpu.VMEM((NS, NB1), jnp.int32),
        pltpu.VMEM_SHARED((NS, NB1), jnp.int32),
        pltpu.SemaphoreType.DMA,
        pltpu.SemaphoreType.DMA,
    ],
)
def _k1(x_hbm, h1_hbm, lh, local, b0, b1, tmp, sh, s0, s1):
    wid = _wid()
    bufs, sems = (b0, b1), (s0, s1)
    pend = [_start_chunk(x_hbm, wid, bufs, sems, 0),
            _start_chunk(x_hbm, wid, bufs, sems, 1)]
    _zero(lh, L * NB1)
    _accum_hist(x_hbm, wid, bufs, sems, pend, lh, NB1, SH1, None, None)
    _lane_reduce(lh, local, NB1)
    _sc_merge_epilogue(local, sh, tmp, h1_hbm, NB1)


@functools.partial(
    pl.kernel,
    out_type=(
        jax.ShapeDtypeStruct((2 * NB2,), jnp.int32),
        jax.ShapeDtypeStruct((L,), jnp.int32),
    ),
    mesh=_mesh,
    compiler_params=_params,
    scratch_types=[
        pltpu.VMEM((2 * NB1,), jnp.int32),
        pltpu.VMEM((L * NB2,), jnp.int32),
        pltpu.VMEM((NB2,), jnp.int32),
        pltpu.VMEM((RB, CW), jnp.float32),
        pltpu.VMEM((RB, CW), jnp.float32),
        pltpu.VMEM((L,), jnp.int32),
        pltpu.VMEM((NS, NB2), jnp.int32),
        pltpu.VMEM_SHARED((NS, NB2), jnp.int32),
        pltpu.SemaphoreType.DMA,
        pltpu.SemaphoreType.DMA,
    ],
)
def _k2(x_hbm, h1_hbm, h2_hbm, sel_hbm, h1buf, lh, local, b0, b1, selbuf,
        tmp, sh, s0, s1):
    wid = _wid()
    bufs, sems = (b0, b1), (s0, s1)
    pend = [_start_chunk(x_hbm, wid, bufs, sems, 0),
            _start_chunk(x_hbm, wid, bufs, sems, 1)]
    pltpu.sync_copy(h1_hbm, h1buf)
    p1, r1 = _select(h1buf, NB1, jnp.int32(RANK))
    _zero(lh, L * NB2)
    _accum_hist(x_hbm, wid, bufs, sems, pend, lh, NB2, SH2, SH1, p1)
    _lane_reduce(lh, local, NB2)
    _sc_merge_epilogue(local, sh, tmp, h2_hbm, NB2)

    @pl.when(wid == 0)
    def _():
        _write_pair(selbuf, sel_hbm, p1, r1)


@functools.partial(
    pl.kernel,
    out_type=(
        jax.ShapeDtypeStruct((2 * NB3,), jnp.int32),
        jax.ShapeDtypeStruct((L,), jnp.int32),
    ),
    mesh=_mesh,
    compiler_params=_params,
    scratch_types=[
        pltpu.VMEM((2 * NB2,), jnp.int32),
        pltpu.VMEM((L * NB3,), jnp.int32),
        pltpu.VMEM((NB3,), jnp.int32),
        pltpu.VMEM((RB, CW), jnp.float32),
        pltpu.VMEM((RB, CW), jnp.float32),
        pltpu.VMEM((L,), jnp.int32),
        pltpu.VMEM((NS, NB3), jnp.int32),
        pltpu.VMEM_SHARED((NS, NB3), jnp.int32),
        pltpu.SemaphoreType.DMA,
        pltpu.SemaphoreType.DMA,
    ],
)
def _k3(x_hbm, h2_hbm, sel1_hbm, h3_hbm, sel2_hbm, h2buf, lh, local, b0, b1,
        selbuf, tmp, sh, s0, s1):
    wid = _wid()
    bufs, sems = (b0, b1), (s0, s1)
    pend = [_start_chunk(x_hbm, wid, bufs, sems, 0),
            _start_chunk(x_hbm, wid, bufs, sems, 1)]
    pltpu.sync_copy(sel1_hbm, selbuf)
    p1, r1 = _read_pair(selbuf)
    pltpu.sync_copy(h2_hbm, h2buf)
    p2, r2 = _select(h2buf, NB2, r1)
    prefix21 = (p1 << 10) | p2
    _zero(lh, L * NB3)
    _accum_hist(x_hbm, wid, bufs, sems, pend, lh, NB3, SH3, SH2, prefix21)
    _lane_reduce(lh, local, NB3)
    _sc_merge_epilogue(local, sh, tmp, h3_hbm, NB3)

    @pl.when(wid == 0)
    def _():
        _write_pair(selbuf, sel2_hbm, prefix21, r2)


def _mask_body(sel_ref, x_ref, h3_ref, tri_ref, o_ref, tscr):
    @pl.when(pl.program_id(0) == 0)
    def _():
        m = jnp.sum(h3_ref[...].astype(jnp.float32), axis=0, keepdims=True)
        exc = jnp.dot(m, tri_ref[...], preferred_element_type=jnp.float32)
        inc = exc + m
        r2 = sel_ref[0, 1].astype(jnp.float32)
        pred = (exc <= r2) & (inc > r2)
        bidx = lax.broadcasted_iota(jnp.int32, (1, NB3), 1)
        p3 = jnp.sum(jnp.where(pred, bidx, 0))
        tscr[0, 0] = (sel_ref[0, 0] << 10) | p3

    t = tscr[0, 0]
    bits = lax.bitcast_convert_type(x_ref[...], jnp.int32)
    u = bits & MASK31
    o_ref[...] = jnp.where(u > t, x_ref[...], 0.0)


def _apply_mask(x, h3, sel2):
    rows = x.shape[0]
    rb = 16
    tri = (jnp.arange(NB3, dtype=jnp.int32)[:, None]
           < jnp.arange(NB3, dtype=jnp.int32)[None, :]).astype(jnp.float32)
    return pl.pallas_call(
        _mask_body,
        grid=(rows // rb,),
        in_specs=[
            pl.BlockSpec(memory_space=pltpu.SMEM),
            pl.BlockSpec((rb, x.shape[1]), lambda i: (i, 0)),
            pl.BlockSpec((2, NB3), lambda i: (0, 0)),
            pl.BlockSpec((NB3, NB3), lambda i: (0, 0)),
        ],
        out_specs=pl.BlockSpec((rb, x.shape[1]), lambda i: (i, 0)),
        out_shape=jax.ShapeDtypeStruct(x.shape, x.dtype),
        scratch_shapes=[pltpu.SMEM((1, 1), jnp.int32)],
    )(sel2.reshape(1, L), x, h3, tri)


@jax.jit
def kernel(x):
    h1 = _k1(x)
    h2, s1 = _k2(x, h1)
    h3, s2 = _k3(x, h2, s1)
    return _apply_mask(x, h3.reshape(2, NB3), s2)


# speedup vs baseline: 40.4390x; 1.0255x over previous
import functools

import jax
import jax.numpy as jnp
from jax import lax
from jax.experimental import pallas as pl
from jax.experimental.pallas import tpu as pltpu
from jax.experimental.pallas import tpu_sc as plsc

ROWS, COLS = 128, 32768
N = ROWS * COLS
K = N // 2
RANK = K - 1
NC, NS, L = 2, 16, 16
NW = NC * NS
RB = 8
CB = COLS // 2
CW = 2048
NCH = CB // CW
UN = 16
NB1, SH1 = 2048, 20
NB2, SH2 = 1024, 10
NB3, SH3 = 1024, 0
MASK31 = 0x7FFFFFFF

_mesh = plsc.VectorSubcoreMesh(core_axis_name="c", subcore_axis_name="s")
_params = pltpu.CompilerParams(
    needs_layout_passes=False, use_tc_tiling_on_sc=True)


def _lanes():
    return lax.iota(jnp.int32, L)


def _wid():
    return lax.axis_index("s") * NC + lax.axis_index("c")


def _zero(ref, nwords):
    z = jnp.zeros((L,), jnp.int32)

    def body(i, _):
        base = i * (L * UN)
        for j in range(UN):
            ref[pl.ds(base + j * L, L)] = z
        return 0

    lax.fori_loop(0, nwords // (L * UN), body, 0)


def _start_chunk(x_hbm, wid, bufs, sems, ci):
    r0 = (wid // 2) * RB
    c0 = (wid % 2) * CB + ci * CW
    src = x_hbm.at[pl.ds(r0, RB), pl.ds(c0, CW)]
    return pltpu.async_copy(src, bufs[ci % 2], sems[ci % 2])


def _accum_hist(x_hbm, wid, bufs, sems, pend, lh, nb, shift, pred_shift,
                pred_val):
    laneoff = _lanes() * nb
    ones = jnp.ones((L,), jnp.int32)

    def compute(buf):
        def vec(i, _):
            vbase = i * (L * UN)
            for r in range(RB):
                idxs = []
                preds = []
                for j in range(UN):
                    v = buf[r, pl.ds(vbase + j * L, L)]
                    u = lax.bitcast_convert_type(v, jnp.int32) & MASK31
                    b = (u >> shift) & (nb - 1)
                    idxs.append(laneoff + b)
                    if pred_shift is not None:
                        preds.append(
                            (u >> pred_shift) == jnp.broadcast_to(pred_val, (L,)))
                for j in range(UN):
                    if pred_shift is None:
                        plsc.addupdate_scatter(lh, [idxs[j]], ones)
                    else:
                        plsc.addupdate_scatter(lh, [idxs[j]], ones,
                                               mask=preds[j])
            return 0

        lax.fori_loop(0, CW // (L * UN), vec, 0)

    for ci in range(NCH):
        pend[ci % 2].wait()
        compute(bufs[ci % 2])
        if ci + 2 < NCH:
            pend[ci % 2] = _start_chunk(x_hbm, wid, bufs, sems, ci + 2)


def _lane_reduce(lh, local, nb):
    def group(g, _):
        off = g * L
        acc = lh[pl.ds(off, L)]
        for l in range(1, L):
            acc = acc + lh[pl.ds(l * nb + off, L)]
        local[pl.ds(off, L)] = acc
        return 0

    lax.fori_loop(0, nb // L, group, 0)


def _sc_merge_epilogue(local, sh, tmp, out_hbm, nb):
    sid = lax.axis_index("s")
    pltpu.sync_copy(local, sh.at[sid])
    plsc.subcore_barrier()

    @pl.when(sid == 0)
    def _():
        pltpu.sync_copy(sh, tmp)

        def group(g, _):
            off = g * L
            acc = tmp[0, pl.ds(off, L)]
            for t in range(1, NS):
                acc = acc + tmp[t, pl.ds(off, L)]
            local[pl.ds(off, L)] = acc
            return 0

        lax.fori_loop(0, nb // L, group, 0)
        scid = lax.axis_index("c")
        pltpu.sync_copy(local, out_hbm.at[pl.ds(scid * nb, nb)])


def _select(hbuf, nb, rank):
    lane = _lanes()

    def group(g, carry):
        run, pacc, excacc = carry
        off = g * L
        v = hbuf[pl.ds(off, L)] + hbuf[pl.ds(nb + off, L)]
        inc = plsc.cumsum(v) + run
        exc = inc - v
        pred = (exc <= rank) & (inc > rank)
        pacc = pacc + jnp.sum(jnp.where(pred, off + lane, 0))
        excacc = excacc + jnp.sum(jnp.where(pred, exc, 0))
        run = run + jnp.sum(v)
        return run, pacc, excacc

    init = (jnp.int32(0), jnp.int32(0), jnp.int32(0))
    _, p, exc = lax.fori_loop(0, nb // L, group, init)
    return p, rank - exc


def _read_pair(selbuf):
    lane = _lanes()
    v = selbuf[...]
    a = jnp.sum(jnp.where(lane == 0, v, 0))
    b = jnp.sum(jnp.where(lane == 1, v, 0))
    return a, b


def _write_pair(selbuf, out_hbm, a, b):
    lane = _lanes()
    selbuf[...] = jnp.where(lane == 0, a, 0) + jnp.where(lane == 1, b, 0)
    pltpu.sync_copy(selbuf, out_hbm)


@functools.partial(
    pl.kernel,
    out_type=jax.ShapeDtypeStruct((2 * NB1,), jnp.int32),
    mesh=_mesh,
    compiler_params=_params,
    scratch_types=[
        pltpu.VMEM((L * NB1,), jnp.int32),
        pltpu.VMEM((NB1,), jnp.int32),
        pltpu.VMEM((RB, CW), jnp.float32),
        pltpu.VMEM((RB, CW), jnp.float32),
        pltpu.VMEM((NS, NB1), jnp.int32),
        pltpu.VMEM_SHARED((NS, NB1), jnp.int32),
        pltpu.SemaphoreType.DMA,
        pltpu.SemaphoreType.DMA,
    ],
)
def _k1(x_hbm, h1_hbm, lh, local, b0, b1, tmp, sh, s0, s1):
    wid = _wid()
    bufs, sems = (b0, b1), (s0, s1)
    pend = [_start_chunk(x_hbm, wid, bufs, sems, 0),
            _start_chunk(x_hbm, wid, bufs, sems, 1)]
    _zero(lh, L * NB1)
    _accum_hist(x_hbm, wid, bufs, sems, pend, lh, NB1, SH1, None, None)
    _lane_reduce(lh, local, NB1)
    _sc_merge_epilogue(local, sh, tmp, h1_hbm, NB1)


@functools.partial(
    pl.kernel,
    out_type=(
        jax.ShapeDtypeStruct((2 * NB2,), jnp.int32),
        jax.ShapeDtypeStruct((L,), jnp.int32),
    ),
    mesh=_mesh,
    compiler_params=_params,
    scratch_types=[
        pltpu.VMEM((2 * NB1,), jnp.int32),
        pltpu.VMEM((L * NB2,), jnp.int32),
        pltpu.VMEM((NB2,), jnp.int32),
        pltpu.VMEM((RB, CW), jnp.float32),
        pltpu.VMEM((RB, CW), jnp.float32),
        pltpu.VMEM((L,), jnp.int32),
        pltpu.VMEM((NS, NB2), jnp.int32),
        pltpu.VMEM_SHARED((NS, NB2), jnp.int32),
        pltpu.SemaphoreType.DMA,
        pltpu.SemaphoreType.DMA,
    ],
)
def _k2(x_hbm, h1_hbm, h2_hbm, sel_hbm, h1buf, lh, local, b0, b1, selbuf,
        tmp, sh, s0, s1):
    wid = _wid()
    bufs, sems = (b0, b1), (s0, s1)
    pend = [_start_chunk(x_hbm, wid, bufs, sems, 0),
            _start_chunk(x_hbm, wid, bufs, sems, 1)]
    pltpu.sync_copy(h1_hbm, h1buf)
    p1, r1 = _select(h1buf, NB1, jnp.int32(RANK))
    _zero(lh, L * NB2)
    _accum_hist(x_hbm, wid, bufs, sems, pend, lh, NB2, SH2, SH1, p1)
    _lane_reduce(lh, local, NB2)
    _sc_merge_epilogue(local, sh, tmp, h2_hbm, NB2)

    @pl.when(wid == 0)
    def _():
        _write_pair(selbuf, sel_hbm, p1, r1)


@functools.partial(
    pl.kernel,
    out_type=(
        jax.ShapeDtypeStruct((2 * NB3,), jnp.int32),
        jax.ShapeDtypeStruct((L,), jnp.int32),
    ),
    mesh=_mesh,
    compiler_params=_params,
    scratch_types=[
        pltpu.VMEM((2 * NB2,), jnp.int32),
        pltpu.VMEM((L * NB3,), jnp.int32),
        pltpu.VMEM((NB3,), jnp.int32),
        pltpu.VMEM((RB, CW), jnp.float32),
        pltpu.VMEM((RB, CW), jnp.float32),
        pltpu.VMEM((L,), jnp.int32),
        pltpu.VMEM((NS, NB3), jnp.int32),
        pltpu.VMEM_SHARED((NS, NB3), jnp.int32),
        pltpu.SemaphoreType.DMA,
        pltpu.SemaphoreType.DMA,
    ],
)
def _k3(x_hbm, h2_hbm, sel1_hbm, h3_hbm, sel2_hbm, h2buf, lh, local, b0, b1,
        selbuf, tmp, sh, s0, s1):
    wid = _wid()
    bufs, sems = (b0, b1), (s0, s1)
    pend = [_start_chunk(x_hbm, wid, bufs, sems, 0),
            _start_chunk(x_hbm, wid, bufs, sems, 1)]
    pltpu.sync_copy(sel1_hbm, selbuf)
    p1, r1 = _read_pair(selbuf)
    pltpu.sync_copy(h2_hbm, h2buf)
    p2, r2 = _select(h2buf, NB2, r1)
    prefix21 = (p1 << 10) | p2
    _zero(lh, L * NB3)
    _accum_hist(x_hbm, wid, bufs, sems, pend, lh, NB3, SH3, SH2, prefix21)
    _lane_reduce(lh, local, NB3)
    _sc_merge_epilogue(local, sh, tmp, h3_hbm, NB3)

    @pl.when(wid == 0)
    def _():
        _write_pair(selbuf, sel2_hbm, prefix21, r2)


def _mask_body(sel_ref, x_ref, h3_ref, o_ref, tscr):
    @pl.when(pl.program_id(0) == 0)
    def _():
        h = h3_ref[...].astype(jnp.float32)
        m8 = h[0:8, :] + h[8:16, :]
        cc = lax.broadcasted_iota(jnp.int32, (128, 128), 0)
        cc2 = lax.broadcasted_iota(jnp.int32, (128, 128), 1)
        tri_l = jnp.where(cc < cc2, 1.0, 0.0)
        rr = lax.broadcasted_iota(jnp.int32, (8, 8), 0)
        rr2 = lax.broadcasted_iota(jnp.int32, (8, 8), 1)
        tri_r = jnp.where(rr2 < rr, 1.0, 0.0)
        within = jnp.dot(m8, tri_l, preferred_element_type=jnp.float32)
        rowsum = jnp.sum(m8, axis=1, keepdims=True)
        rowexc = jnp.dot(tri_r, rowsum, preferred_element_type=jnp.float32)
        exc = within + rowexc
        inc = exc + m8
        r2 = sel_ref[0, 1].astype(jnp.float32)
        pred = (exc <= r2) & (inc > r2)
        flat = (lax.broadcasted_iota(jnp.int32, (8, 128), 0) * 128
                + lax.broadcasted_iota(jnp.int32, (8, 128), 1))
        p3 = jnp.sum(jnp.where(pred, flat, 0))
        tscr[0, 0] = (sel_ref[0, 0] << 10) | p3

    t = tscr[0, 0]
    bits = lax.bitcast_convert_type(x_ref[...], jnp.int32)
    u = bits & MASK31
    o_ref[...] = jnp.where(u > t, x_ref[...], 0.0)


def _apply_mask(x, h3, sel2):
    rows = x.shape[0]
    rb = 16
    return pl.pallas_call(
        _mask_body,
        grid=(rows // rb,),
        in_specs=[
            pl.BlockSpec(memory_space=pltpu.SMEM),
            pl.BlockSpec((rb, x.shape[1]), lambda i: (i, 0)),
            pl.BlockSpec((16, 128), lambda i: (0, 0)),
        ],
        out_specs=pl.BlockSpec((rb, x.shape[1]), lambda i: (i, 0)),
        out_shape=jax.ShapeDtypeStruct(x.shape, x.dtype),
        scratch_shapes=[pltpu.SMEM((1, 1), jnp.int32)],
    )(sel2.reshape(1, L), x, h3)


@jax.jit
def kernel(x):
    h1 = _k1(x)
    h2, s1 = _k2(x, h1)
    h3, s2 = _k3(x, h2, s1)
    return _apply_mask(x, h3.reshape(16, 128), s2)


# speedup vs baseline: 40.5105x; 1.0018x over previous
import functools

import jax
import jax.numpy as jnp
from jax import lax
from jax.experimental import pallas as pl
from jax.experimental.pallas import tpu as pltpu
from jax.experimental.pallas import tpu_sc as plsc

ROWS, COLS = 128, 32768
N = ROWS * COLS
K = N // 2
RANK = K - 1
NC, NS, L = 2, 16, 16
NW = NC * NS
RB = 8
CB = COLS // 2
CW = 2048
NCH = CB // CW
UN = 16
NB1, SH1 = 2048, 20
NB2, SH2 = 1024, 10
NB3, SH3 = 1024, 0
MASK31 = 0x7FFFFFFF

_mesh = plsc.VectorSubcoreMesh(core_axis_name="c", subcore_axis_name="s")
_params = pltpu.CompilerParams(
    needs_layout_passes=False, use_tc_tiling_on_sc=True)


def _lanes():
    return lax.iota(jnp.int32, L)


def _wid():
    return lax.axis_index("s") * NC + lax.axis_index("c")


def _zero(ref, nwords):
    z = jnp.zeros((L,), jnp.int32)

    def body(i, _):
        base = i * (L * UN)
        for j in range(UN):
            ref[pl.ds(base + j * L, L)] = z
        return 0

    lax.fori_loop(0, nwords // (L * UN), body, 0)


def _start_chunk(x_hbm, wid, bufs, sems, ci):
    r0 = (wid // 2) * RB
    c0 = (wid % 2) * CB + ci * CW
    src = x_hbm.at[pl.ds(r0, RB), pl.ds(c0, CW)]
    return pltpu.async_copy(src, bufs[ci % 2], sems[ci % 2])


def _accum_hist(x_hbm, wid, bufs, sems, pend, lh, nb, shift, pred_shift,
                pred_val):
    laneoff = _lanes() * nb
    ones = jnp.ones((L,), jnp.int32)

    def compute(buf):
        def vec(i, _):
            vbase = i * (L * UN)
            for r in range(RB):
                idxs = []
                preds = []
                for j in range(UN):
                    v = buf[r, pl.ds(vbase + j * L, L)]
                    u = lax.bitcast_convert_type(v, jnp.int32) & MASK31
                    b = (u >> shift) & (nb - 1)
                    idxs.append(laneoff + b)
                    if pred_shift is not None:
                        preds.append(
                            (u >> pred_shift) == jnp.broadcast_to(pred_val, (L,)))
                for j in range(UN):
                    if pred_shift is None:
                        plsc.addupdate_scatter(lh, [idxs[j]], ones)
                    else:
                        plsc.addupdate_scatter(lh, [idxs[j]], ones,
                                               mask=preds[j])
            return 0

        lax.fori_loop(0, CW // (L * UN), vec, 0)

    for ci in range(NCH):
        pend[ci % 2].wait()
        compute(bufs[ci % 2])
        if ci + 2 < NCH:
            pend[ci % 2] = _start_chunk(x_hbm, wid, bufs, sems, ci + 2)


def _lane_reduce(lh, local, nb):
    def group(g, _):
        off = g * L
        acc = lh[pl.ds(off, L)]
        for l in range(1, L):
            acc = acc + lh[pl.ds(l * nb + off, L)]
        local[pl.ds(off, L)] = acc
        return 0

    lax.fori_loop(0, nb // L, group, 0)


def _sc_merge_epilogue(local, sh, tmp, out_hbm, nb, out2d=None):
    sid = lax.axis_index("s")
    pltpu.sync_copy(local, sh.at[sid])
    plsc.subcore_barrier()

    @pl.when(sid == 0)
    def _():
        pltpu.sync_copy(sh, tmp)

        def group(g, _):
            off = g * L
            acc = tmp[0, pl.ds(off, L)]
            for t in range(1, NS):
                acc = acc + tmp[t, pl.ds(off, L)]
            if out2d is not None:
                out2d[off >> 7, pl.ds(off & 127, L)] = acc
            else:
                local[pl.ds(off, L)] = acc
            return 0

        lax.fori_loop(0, nb // L, group, 0)
        scid = lax.axis_index("c")
        if out2d is not None:
            pltpu.sync_copy(
                out2d, out_hbm.at[pl.ds(scid * (nb // 128), nb // 128), :])
        else:
            pltpu.sync_copy(local, out_hbm.at[pl.ds(scid * nb, nb)])


def _select(hbuf, nb, rank):
    lane = _lanes()

    def group(g, carry):
        run, pacc, excacc = carry
        off = g * L
        v = hbuf[pl.ds(off, L)] + hbuf[pl.ds(nb + off, L)]
        inc = plsc.cumsum(v) + run
        exc = inc - v
        pred = (exc <= rank) & (inc > rank)
        pacc = pacc + jnp.sum(jnp.where(pred, off + lane, 0))
        excacc = excacc + jnp.sum(jnp.where(pred, exc, 0))
        run = run + jnp.sum(v)
        return run, pacc, excacc

    init = (jnp.int32(0), jnp.int32(0), jnp.int32(0))
    _, p, exc = lax.fori_loop(0, nb // L, group, init)
    return p, rank - exc


def _read_pair(selbuf):
    lane = _lanes()
    v = selbuf[...]
    a = jnp.sum(jnp.where(lane == 0, v, 0))
    b = jnp.sum(jnp.where(lane == 1, v, 0))
    return a, b


def _write_pair(selbuf, out_hbm, a, b):
    lane = _lanes()
    selbuf[...] = jnp.where(lane == 0, a, 0) + jnp.where(lane == 1, b, 0)
    pltpu.sync_copy(selbuf, out_hbm)


@functools.partial(
    pl.kernel,
    out_type=jax.ShapeDtypeStruct((2 * NB1,), jnp.int32),
    mesh=_mesh,
    compiler_params=_params,
    scratch_types=[
        pltpu.VMEM((L * NB1,), jnp.int32),
        pltpu.VMEM((NB1,), jnp.int32),
        pltpu.VMEM((RB, CW), jnp.float32),
        pltpu.VMEM((RB, CW), jnp.float32),
        pltpu.VMEM((NS, NB1), jnp.int32),
        pltpu.VMEM_SHARED((NS, NB1), jnp.int32),
        pltpu.SemaphoreType.DMA,
        pltpu.SemaphoreType.DMA,
    ],
)
def _k1(x_hbm, h1_hbm, lh, local, b0, b1, tmp, sh, s0, s1):
    wid = _wid()
    bufs, sems = (b0, b1), (s0, s1)
    pend = [_start_chunk(x_hbm, wid, bufs, sems, 0),
            _start_chunk(x_hbm, wid, bufs, sems, 1)]
    _zero(lh, L * NB1)
    _accum_hist(x_hbm, wid, bufs, sems, pend, lh, NB1, SH1, None, None)
    _lane_reduce(lh, local, NB1)
    _sc_merge_epilogue(local, sh, tmp, h1_hbm, NB1)


@functools.partial(
    pl.kernel,
    out_type=(
        jax.ShapeDtypeStruct((2 * NB2,), jnp.int32),
        jax.ShapeDtypeStruct((L,), jnp.int32),
    ),
    mesh=_mesh,
    compiler_params=_params,
    scratch_types=[
        pltpu.VMEM((2 * NB1,), jnp.int32),
        pltpu.VMEM((L * NB2,), jnp.int32),
        pltpu.VMEM((NB2,), jnp.int32),
        pltpu.VMEM((RB, CW), jnp.float32),
        pltpu.VMEM((RB, CW), jnp.float32),
        pltpu.VMEM((L,), jnp.int32),
        pltpu.VMEM((NS, NB2), jnp.int32),
        pltpu.VMEM_SHARED((NS, NB2), jnp.int32),
        pltpu.SemaphoreType.DMA,
        pltpu.SemaphoreType.DMA,
    ],
)
def _k2(x_hbm, h1_hbm, h2_hbm, sel_hbm, h1buf, lh, local, b0, b1, selbuf,
        tmp, sh, s0, s1):
    wid = _wid()
    bufs, sems = (b0, b1), (s0, s1)
    pend = [_start_chunk(x_hbm, wid, bufs, sems, 0),
            _start_chunk(x_hbm, wid, bufs, sems, 1)]
    pltpu.sync_copy(h1_hbm, h1buf)
    p1, r1 = _select(h1buf, NB1, jnp.int32(RANK))
    _zero(lh, L * NB2)
    _accum_hist(x_hbm, wid, bufs, sems, pend, lh, NB2, SH2, SH1, p1)
    _lane_reduce(lh, local, NB2)
    _sc_merge_epilogue(local, sh, tmp, h2_hbm, NB2)

    @pl.when(wid == 0)
    def _():
        _write_pair(selbuf, sel_hbm, p1, r1)


@functools.partial(
    pl.kernel,
    out_type=(
        jax.ShapeDtypeStruct((16, 128), jnp.int32),
        jax.ShapeDtypeStruct((L,), jnp.int32),
    ),
    mesh=_mesh,
    compiler_params=_params,
    scratch_types=[
        pltpu.VMEM((2 * NB2,), jnp.int32),
        pltpu.VMEM((L * NB3,), jnp.int32),
        pltpu.VMEM((NB3,), jnp.int32),
        pltpu.VMEM((RB, CW), jnp.float32),
        pltpu.VMEM((RB, CW), jnp.float32),
        pltpu.VMEM((L,), jnp.int32),
        pltpu.VMEM((NS, NB3), jnp.int32),
        pltpu.VMEM_SHARED((NS, NB3), jnp.int32),
        pltpu.VMEM((8, 128), jnp.int32),
        pltpu.SemaphoreType.DMA,
        pltpu.SemaphoreType.DMA,
    ],
)
def _k3(x_hbm, h2_hbm, sel1_hbm, h3_hbm, sel2_hbm, h2buf, lh, local, b0, b1,
        selbuf, tmp, sh, out2d, s0, s1):
    wid = _wid()
    bufs, sems = (b0, b1), (s0, s1)
    pend = [_start_chunk(x_hbm, wid, bufs, sems, 0),
            _start_chunk(x_hbm, wid, bufs, sems, 1)]
    pltpu.sync_copy(sel1_hbm, selbuf)
    p1, r1 = _read_pair(selbuf)
    pltpu.sync_copy(h2_hbm, h2buf)
    p2, r2 = _select(h2buf, NB2, r1)
    prefix21 = (p1 << 10) | p2
    _zero(lh, L * NB3)
    _accum_hist(x_hbm, wid, bufs, sems, pend, lh, NB3, SH3, SH2, prefix21)
    _lane_reduce(lh, local, NB3)
    _sc_merge_epilogue(local, sh, tmp, h3_hbm, NB3, out2d=out2d)

    @pl.when(wid == 0)
    def _():
        _write_pair(selbuf, sel2_hbm, prefix21, r2)


def _mask_body(sel_ref, x_ref, h3_ref, o_ref, tscr):
    @pl.when(pl.program_id(0) == 0)
    def _():
        h = h3_ref[...].astype(jnp.float32)
        m8 = h[0:8, :] + h[8:16, :]
        cc = lax.broadcasted_iota(jnp.int32, (128, 128), 0)
        cc2 = lax.broadcasted_iota(jnp.int32, (128, 128), 1)
        tri_l = jnp.where(cc < cc2, 1.0, 0.0)
        rr = lax.broadcasted_iota(jnp.int32, (8, 8), 0)
        rr2 = lax.broadcasted_iota(jnp.int32, (8, 8), 1)
        tri_r = jnp.where(rr2 < rr, 1.0, 0.0)
        within = jnp.dot(m8, tri_l, preferred_element_type=jnp.float32)
        rowsum = jnp.sum(m8, axis=1, keepdims=True)
        rowexc = jnp.dot(tri_r, rowsum, preferred_element_type=jnp.float32)
        exc = within + rowexc
        inc = exc + m8
        r2 = sel_ref[0, 1].astype(jnp.float32)
        pred = (exc <= r2) & (inc > r2)
        flat = (lax.broadcasted_iota(jnp.int32, (8, 128), 0) * 128
                + lax.broadcasted_iota(jnp.int32, (8, 128), 1))
        p3 = jnp.sum(jnp.where(pred, flat, 0))
        tscr[0, 0] = (sel_ref[0, 0] << 10) | p3

    t = tscr[0, 0]
    bits = lax.bitcast_convert_type(x_ref[...], jnp.int32)
    u = bits & MASK31
    o_ref[...] = jnp.where(u > t, x_ref[...], 0.0)


def _apply_mask(x, h3, sel2):
    rows = x.shape[0]
    rb = 16
    return pl.pallas_call(
        _mask_body,
        grid=(rows // rb,),
        in_specs=[
            pl.BlockSpec(memory_space=pltpu.SMEM),
            pl.BlockSpec((rb, x.shape[1]), lambda i: (i, 0)),
            pl.BlockSpec((16, 128), lambda i: (0, 0)),
        ],
        out_specs=pl.BlockSpec((rb, x.shape[1]), lambda i: (i, 0)),
        out_shape=jax.ShapeDtypeStruct(x.shape, x.dtype),
        scratch_shapes=[pltpu.SMEM((1, 1), jnp.int32)],
    )(sel2.reshape(1, L), x, h3)


@jax.jit
def kernel(x):
    h1 = _k1(x)
    h2, s1 = _k2(x, h1)
    h3, s2 = _k3(x, h2, s1)
    return _apply_mask(x, h3, s2)
